# Initial kernel scaffold; baseline (speedup 1.0000x reference)
import jax
import jax.numpy as jnp
from jax.experimental import pallas as pl


def kernel(inp, data):
    raise NotImplementedError("write your pallas kernel here")


# speedup vs baseline: 1.9987x; 1.9987x over previous
import functools

import jax
import jax.numpy as jnp
import numpy as np
from jax import lax
from jax.experimental import pallas as pl
from jax.experimental.pallas import tpu as pltpu
from jax.experimental.pallas import tpu_sc as plsc

B = 16384
L = 200
TABLE = 1024
N = B * L

NUM_CORES = 2
NUM_SUBCORES = 16
NW = NUM_CORES * NUM_SUBCORES
PER_W = N // NW

CHUNK = 4096
ROWS = CHUNK // 128
GROUPS = CHUNK // 16
N_CHUNKS = PER_W // CHUNK

B0 = np.float32(1.2)
B1 = np.float32(0.07)
M0 = np.float32(1023.9999 / (0.6 - (-1.2)))
M1 = np.float32(1023.9999 / (2 * 0.07))


def _sc_kernel(inp_hbm, tab_hbm, out_hbm, inp_v, idx_v, val_v, gsem):
    wid = lax.axis_index("s") * NUM_CORES + lax.axis_index("c")
    wbase = wid * PER_W
    iota = lax.iota(jnp.int32, 16)

    def chunk_body(c, carry):
        base = wbase + c * CHUNK
        pltpu.sync_copy(inp_hbm.at[pl.ds(2 * base, 2 * CHUNK)], inp_v)

        def grp(g, carry2):
            off = 32 * g
            ex = plsc.load_gather(inp_v, [off + 2 * iota])
            ey = plsc.load_gather(inp_v, [off + 2 * iota + 1])
            xi = ((ex + B0) * M0).astype(jnp.int32)
            yi = ((ey + B1) * M1).astype(jnp.int32)
            idx_v[g // 8, pl.ds((g % 8) * 16, 16)] = xi * TABLE + yi
            return carry2

        lax.fori_loop(0, GROUPS, grp, 0, unroll=4)

        descs = []
        for j in range(ROWS):
            descs.append(
                pltpu.async_copy(
                    tab_hbm.at[idx_v.at[j]],
                    val_v.at[pl.ds(j * 128, 128)],
                    gsem,
                )
            )
        for d in descs:
            d.wait()

        pltpu.sync_copy(val_v, out_hbm.at[pl.ds(base, CHUNK)])
        return carry

    lax.fori_loop(0, N_CHUNKS, chunk_body, 0)


@jax.jit
def kernel(inp, data):
    mesh = plsc.VectorSubcoreMesh(core_axis_name="c", subcore_axis_name="s")
    run = functools.partial(
        pl.kernel,
        mesh=mesh,
        compiler_params=pltpu.CompilerParams(needs_layout_passes=False),
        out_type=jax.ShapeDtypeStruct((N,), jnp.float32),
        scratch_types=[
            pltpu.VMEM((2 * CHUNK,), jnp.float32),
            pltpu.VMEM((ROWS, 128), jnp.int32),
            pltpu.VMEM((CHUNK,), jnp.float32),
            pltpu.SemaphoreType.DMA,
        ],
    )(_sc_kernel)
    out = run(inp.reshape(-1), data.reshape(-1))
    return out.reshape(B, L)


# speedup vs baseline: 7.5535x; 3.7792x over previous
import functools

import jax
import jax.numpy as jnp
import numpy as np
from jax import lax
from jax.experimental import pallas as pl
from jax.experimental.pallas import tpu as pltpu
from jax.experimental.pallas import tpu_sc as plsc

B = 16384
L = 200
TABLE = 1024
N = B * L

NUM_CORES = 2
NUM_SUBCORES = 16
NW = NUM_CORES * NUM_SUBCORES
PER_W = N // NW

CHUNK = 4096
ROWS = CHUNK // 128
GROUPS = CHUNK // 16
N_CHUNKS = PER_W // CHUNK

B0 = np.float32(1.2)
B1 = np.float32(0.07)
M0 = np.float32(1023.9999 / (0.6 - (-1.2)))
M1 = np.float32(1023.9999 / (2 * 0.07))


def _sc_kernel(inp_hbm, tab_hbm, out_hbm, tab_sh, inp_v, idx_v, val_v, gsem):
    cid = lax.axis_index("c")
    sid = lax.axis_index("s")
    wid = sid * NUM_CORES + cid
    wbase = wid * PER_W
    iota = lax.iota(jnp.int32, 16)

    SHARD = (TABLE * TABLE) // NUM_SUBCORES
    pltpu.sync_copy(
        tab_hbm.at[pl.ds(sid * SHARD, SHARD)],
        tab_sh.at[pl.ds(sid * SHARD, SHARD)],
    )
    plsc.subcore_barrier()

    def chunk_body(c, carry):
        base = wbase + c * CHUNK
        pltpu.sync_copy(inp_hbm.at[pl.ds(2 * base, 2 * CHUNK)], inp_v)

        def grp(g, carry2):
            off = 32 * g
            ex = plsc.load_gather(inp_v, [off + 2 * iota])
            ey = plsc.load_gather(inp_v, [off + 2 * iota + 1])
            xi = ((ex + B0) * M0).astype(jnp.int32)
            yi = ((ey + B1) * M1).astype(jnp.int32)
            idx_v[g // 8, pl.ds((g % 8) * 16, 16)] = xi * TABLE + yi
            return carry2

        lax.fori_loop(0, GROUPS, grp, 0, unroll=4)

        descs = []
        for j in range(ROWS):
            descs.append(
                pltpu.async_copy(
                    tab_sh.at[idx_v.at[j]],
                    val_v.at[pl.ds(j * 128, 128)],
                    gsem,
                )
            )
        for d in descs:
            d.wait()

        pltpu.sync_copy(val_v, out_hbm.at[pl.ds(base, CHUNK)])
        return carry

    lax.fori_loop(0, N_CHUNKS, chunk_body, 0)


@jax.jit
def kernel(inp, data):
    mesh = plsc.VectorSubcoreMesh(core_axis_name="c", subcore_axis_name="s")
    run = functools.partial(
        pl.kernel,
        mesh=mesh,
        compiler_params=pltpu.CompilerParams(needs_layout_passes=False),
        out_type=jax.ShapeDtypeStruct((N,), jnp.float32),
        scratch_types=[
            pltpu.VMEM_SHARED((TABLE * TABLE,), jnp.float32),
            pltpu.VMEM((2 * CHUNK,), jnp.float32),
            pltpu.VMEM((ROWS, 128), jnp.int32),
            pltpu.VMEM((CHUNK,), jnp.float32),
            pltpu.SemaphoreType.DMA,
        ],
    )(_sc_kernel)
    out = run(inp.reshape(-1), data.reshape(-1))
    return out.reshape(B, L)


# speedup vs baseline: 38.3503x; 5.0772x over previous
import functools

import jax
import jax.numpy as jnp
import numpy as np
from jax import lax
from jax.experimental import pallas as pl
from jax.experimental.pallas import tpu as pltpu
from jax.experimental.pallas import tpu_sc as plsc

B = 16384
L = 200
TABLE = 1024
N = B * L

NUM_CORES = 2
NUM_SUBCORES = 16
NW = NUM_CORES * NUM_SUBCORES

NJT = L // 8
NIT = B // 128
IT_PER_W = NIT // NW
CHUNK = IT_PER_W * 1024
ROWS = CHUNK // 128
GROUPS = CHUNK // 16

B0 = np.float32(1.2)
B1 = np.float32(0.07)
M0 = np.float32(1023.9999 / (0.6 - (-1.2)))
M1 = np.float32(1023.9999 / (2 * 0.07))


def _sc_kernel(inp_hbm, tab_hbm, out_hbm, tab_sh, inp_v, idx_v, val_v, gsem):
    cid = lax.axis_index("c")
    sid = lax.axis_index("s")
    wid = sid * NUM_CORES + cid
    it0 = wid * IT_PER_W

    SHARD = (TABLE * TABLE) // NUM_SUBCORES
    pltpu.sync_copy(
        tab_hbm.at[pl.ds(sid * SHARD, SHARD)],
        tab_sh.at[pl.ds(sid * SHARD, SHARD)],
    )
    plsc.subcore_barrier()

    def chunk_body(jt, carry):
        pltpu.sync_copy(
            inp_hbm.at[pl.ds(jt * 8, 8), pl.ds(it0 * 256, IT_PER_W * 256)],
            inp_v,
        )

        def grp(g, carry2):
            itl = g // 64
            js = (g // 8) % 8
            col = (g % 8) * 16
            x = inp_v[js, pl.ds(itl * 256 + col, 16)]
            y = inp_v[js, pl.ds(itl * 256 + 128 + col, 16)]
            xi = ((x + B0) * M0).astype(jnp.int32)
            yi = ((y + B1) * M1).astype(jnp.int32)
            idx_v[g // 8, pl.ds(col, 16)] = xi * TABLE + yi
            return carry2

        lax.fori_loop(0, GROUPS, grp, 0, unroll=4)

        descs = []
        for j in range(ROWS):
            descs.append(
                pltpu.async_copy(
                    tab_sh.at[idx_v.at[j]],
                    val_v.at[pl.ds(j * 128, 128)],
                    gsem,
                )
            )
        for d in descs:
            d.wait()

        pltpu.sync_copy(val_v, out_hbm.at[jt, pl.ds(it0 * 1024, CHUNK)])
        return carry

    lax.fori_loop(0, NJT, chunk_body, 0)


@jax.jit
def kernel(inp, data):
    mesh = plsc.VectorSubcoreMesh(core_axis_name="c", subcore_axis_name="s")
    run = functools.partial(
        pl.kernel,
        mesh=mesh,
        compiler_params=pltpu.CompilerParams(needs_layout_passes=False),
        out_type=jax.ShapeDtypeStruct((NJT, NIT * 1024), jnp.float32),
        scratch_types=[
            pltpu.VMEM_SHARED((TABLE * TABLE,), jnp.float32),
            pltpu.VMEM((8, IT_PER_W * 256), jnp.float32),
            pltpu.VMEM((ROWS, 128), jnp.int32),
            pltpu.VMEM((CHUNK,), jnp.float32),
            pltpu.SemaphoreType.DMA,
        ],
    )(_sc_kernel)
    inp_phys = (
        inp.reshape(NIT, 128, L, 2).transpose(2, 0, 3, 1).reshape(L, NIT * 256)
    )
    out_phys = run(inp_phys, data.reshape(-1))
    return (
        out_phys.reshape(NJT, NIT, 8, 128).transpose(1, 3, 0, 2).reshape(B, L)
    )


# speedup vs baseline: 38.5170x; 1.0043x over previous
import functools

import jax
import jax.numpy as jnp
import numpy as np
from jax import lax
from jax.experimental import pallas as pl
from jax.experimental.pallas import tpu as pltpu
from jax.experimental.pallas import tpu_sc as plsc

B = 16384
L = 200
TABLE = 1024
N = B * L

NUM_CORES = 2
NUM_SUBCORES = 16
NW = NUM_CORES * NUM_SUBCORES

NJT = L // 8
NIT = B // 128
IT_PER_W = NIT // NW
CHUNK = IT_PER_W * 1024
ROWS = CHUNK // 128
GROUPS = CHUNK // 16

B0 = np.float32(1.2)
B1 = np.float32(0.07)
M0 = np.float32(1023.9999 / (0.6 - (-1.2)))
M1 = np.float32(1023.9999 / (2 * 0.07))


def _sc_kernel(
    inp_hbm, tab_hbm, out_hbm, tab_sh, inp_v, idx_v, val_v, isem, gsem, osem
):
    cid = lax.axis_index("c")
    sid = lax.axis_index("s")
    wid = sid * NUM_CORES + cid
    it0 = wid * IT_PER_W

    SHARD = (TABLE * TABLE) // NUM_SUBCORES
    pltpu.sync_copy(
        tab_hbm.at[pl.ds(sid * SHARD, SHARD)],
        tab_sh.at[pl.ds(sid * SHARD, SHARD)],
    )
    plsc.subcore_barrier()

    def inp_src(c):
        return inp_hbm.at[pl.ds(c * 8, 8), pl.ds(it0 * 256, IT_PER_W * 256)]

    def out_dst(c):
        return out_hbm.at[c, pl.ds(it0 * 1024, CHUNK)]

    def start_inp(c):
        pltpu.async_copy(inp_src(c), inp_v.at[c & 1], isem)

    def wait_inp(c):
        pltpu.make_async_copy(inp_src(c), inp_v.at[c & 1], isem).wait()

    def compute(c):
        p = c & 1

        def grp(g, carry2):
            itl = g // 64
            js = (g // 8) % 8
            col = (g % 8) * 16
            x = inp_v[p, js, pl.ds(itl * 256 + col, 16)]
            y = inp_v[p, js, pl.ds(itl * 256 + 128 + col, 16)]
            xi = ((x + B0) * M0).astype(jnp.int32)
            yi = ((y + B1) * M1).astype(jnp.int32)
            idx_v[p, g // 8, pl.ds(col, 16)] = xi * TABLE + yi
            return carry2

        lax.fori_loop(0, GROUPS, grp, 0, unroll=4)

    def fire_gathers(c):
        p = c & 1
        for j in range(ROWS):
            pltpu.async_copy(
                tab_sh.at[idx_v.at[p, j]],
                val_v.at[p, pl.ds(j * 128, 128)],
                gsem,
            )

    def wait_gathers(c):
        pltpu.make_async_copy(out_dst(c), val_v.at[c & 1], gsem).wait()

    def start_out(c):
        pltpu.async_copy(val_v.at[c & 1], out_dst(c), osem)

    def wait_out(c):
        pltpu.make_async_copy(val_v.at[c & 1], out_dst(c), osem).wait()

    start_inp(0)

    def chunk_body(c, carry):
        pl.when(c < NJT)(lambda: wait_inp(c))
        pl.when(c + 1 < NJT)(lambda: start_inp(c + 1))
        pl.when(c < NJT)(lambda: compute(c))
        pl.when((c >= 1) & (c <= NJT))(lambda: wait_gathers(c - 1))
        pl.when((c >= 1) & (c <= NJT))(lambda: start_out(c - 1))
        pl.when((c >= 2) & (c <= NJT + 1))(lambda: wait_out(c - 2))
        pl.when(c < NJT)(lambda: fire_gathers(c))
        return carry

    lax.fori_loop(0, NJT + 2, chunk_body, 0)


@jax.jit
def kernel(inp, data):
    mesh = plsc.VectorSubcoreMesh(core_axis_name="c", subcore_axis_name="s")
    run = functools.partial(
        pl.kernel,
        mesh=mesh,
        compiler_params=pltpu.CompilerParams(needs_layout_passes=False),
        out_type=jax.ShapeDtypeStruct((NJT, NIT * 1024), jnp.float32),
        scratch_types=[
            pltpu.VMEM_SHARED((TABLE * TABLE,), jnp.float32),
            pltpu.VMEM((2, 8, IT_PER_W * 256), jnp.float32),
            pltpu.VMEM((2, ROWS, 128), jnp.int32),
            pltpu.VMEM((2, CHUNK), jnp.float32),
            pltpu.SemaphoreType.DMA,
            pltpu.SemaphoreType.DMA,
            pltpu.SemaphoreType.DMA,
        ],
    )(_sc_kernel)
    inp_phys = (
        inp.reshape(NIT, 128, L, 2).transpose(2, 0, 3, 1).reshape(L, NIT * 256)
    )
    out_phys = run(inp_phys, data.reshape(-1))
    return (
        out_phys.reshape(NJT, NIT, 8, 128).transpose(1, 3, 0, 2).reshape(B, L)
    )


# speedup vs baseline: 179.0964x; 4.6498x over previous
import functools

import jax
import jax.numpy as jnp
import numpy as np
from jax import lax
from jax.experimental import pallas as pl
from jax.experimental.pallas import tpu as pltpu
from jax.experimental.pallas import tpu_sc as plsc

B = 16384
L = 200
TABLE = 1024
N = B * L

NUM_CORES = 2
NUM_SUBCORES = 16
NW = NUM_CORES * NUM_SUBCORES

NJT = L // 8
NIT = B // 128
IT_PER_W = NIT // NW
CHUNK = IT_PER_W * 1024
ROWS = CHUNK // 128
GROUPS = CHUNK // 16

NTAB = TABLE * TABLE
WINDOW = 32768
I32MIN = np.int32(-2147483648)
I32MAX = np.int32(2147483647)

B0 = np.float32(1.2)
B1 = np.float32(0.07)
M0 = np.float32(1023.9999 / (0.6 - (-1.2)))
M1 = np.float32(1023.9999 / (2 * 0.07))


def _sc_kernel(
    inp_hbm, tab_hbm, out_hbm, tab_sh, inp_v, idx_v, val_v, cache_v,
    mflag, wstate, used_dma, isem, gsem, osem
):
    cid = lax.axis_index("c")
    sid = lax.axis_index("s")
    wid = sid * NUM_CORES + cid
    it0 = wid * IT_PER_W

    SHARD = NTAB // NUM_SUBCORES
    pltpu.sync_copy(
        tab_hbm.at[pl.ds(sid * SHARD, SHARD)],
        tab_sh.at[pl.ds(sid * SHARD, SHARD)],
    )
    wstate[0] = 0
    wstate[1] = 0
    plsc.subcore_barrier()

    def inp_src(c):
        return inp_hbm.at[pl.ds(c * 8, 8), pl.ds(it0 * 256, IT_PER_W * 256)]

    def out_dst(c):
        return out_hbm.at[c, pl.ds(it0 * 1024, CHUNK)]

    def start_inp(c):
        pltpu.async_copy(inp_src(c), inp_v.at[c & 1], isem)

    def wait_inp(c):
        pltpu.make_async_copy(inp_src(c), inp_v.at[c & 1], isem).wait()

    def compute(c):
        p = c & 1

        def grp(g, carry2):
            mn, mx = carry2
            itl = g // 64
            js = (g // 8) % 8
            col = (g % 8) * 16
            x = inp_v[p, js, pl.ds(itl * 256 + col, 16)]
            y = inp_v[p, js, pl.ds(itl * 256 + 128 + col, 16)]
            xi = ((x + B0) * M0).astype(jnp.int32)
            yi = ((y + B1) * M1).astype(jnp.int32)
            iv = xi * TABLE + yi
            idx_v[p, g // 8, pl.ds(col, 16)] = iv
            return jnp.minimum(mn, iv), jnp.maximum(mx, iv)

        mn, mx = lax.fori_loop(
            0,
            GROUPS,
            grp,
            (jnp.full((16,), I32MAX, jnp.int32),
             jnp.full((16,), I32MIN, jnp.int32)),
            unroll=4,
        )
        mflag[2 * p] = -lax.reduce_max(-mn, (0,))
        mflag[2 * p + 1] = lax.reduce_max(mx, (0,))

    def gather_chunk(c):
        p = c & 1
        mn = mflag[2 * p]
        mx = mflag[2 * p + 1]
        base_cur = wstate[0]
        fits_cur = jnp.logical_and(
            wstate[1] == 1,
            jnp.logical_and(mn >= base_cur, mx < base_cur + WINDOW),
        )
        new_base = pl.multiple_of(
            jnp.clip((mn >> 3) << 3, 0, NTAB - WINDOW), 8
        )
        fits_new = jnp.logical_and(mn >= new_base, mx < new_base + WINDOW)

        def restage():
            pltpu.sync_copy(tab_sh.at[pl.ds(new_base, WINDOW)], cache_v)
            wstate[0] = new_base
            wstate[1] = 1

        pl.when(jnp.logical_and(jnp.logical_not(fits_cur), fits_new))(restage)

        def local_gather():
            b = wstate[0]

            def lg(g, carry2):
                col = (g % 8) * 16
                iv = idx_v[p, g // 8, pl.ds(col, 16)] - b
                val_v[p, pl.ds(g * 16, 16)] = plsc.load_gather(cache_v, [iv])
                return carry2

            lax.fori_loop(0, GROUPS, lg, 0, unroll=4)
            used_dma[p] = 0

        def dma_gather():
            for j in range(ROWS):
                pltpu.async_copy(
                    tab_sh.at[idx_v.at[p, j]],
                    val_v.at[p, pl.ds(j * 128, 128)],
                    gsem,
                )
            used_dma[p] = 1

        use_local = jnp.logical_or(fits_cur, fits_new)
        pl.when(use_local)(local_gather)
        pl.when(jnp.logical_not(use_local))(dma_gather)

    def wait_gathers(c):
        pl.when(used_dma[c & 1] == 1)(
            lambda: pltpu.make_async_copy(
                out_dst(c), val_v.at[c & 1], gsem
            ).wait()
        )

    def start_out(c):
        pltpu.async_copy(val_v.at[c & 1], out_dst(c), osem)

    def wait_out(c):
        pltpu.make_async_copy(val_v.at[c & 1], out_dst(c), osem).wait()

    start_inp(0)

    def chunk_body(c, carry):
        pl.when(c < NJT)(lambda: wait_inp(c))
        pl.when(c + 1 < NJT)(lambda: start_inp(c + 1))
        pl.when(c < NJT)(lambda: compute(c))
        pl.when((c >= 1) & (c <= NJT))(lambda: wait_gathers(c - 1))
        pl.when((c >= 1) & (c <= NJT))(lambda: start_out(c - 1))
        pl.when((c >= 2) & (c <= NJT + 1))(lambda: wait_out(c - 2))
        pl.when(c < NJT)(lambda: gather_chunk(c))
        return carry

    lax.fori_loop(0, NJT + 2, chunk_body, 0)


@jax.jit
def kernel(inp, data):
    mesh = plsc.VectorSubcoreMesh(core_axis_name="c", subcore_axis_name="s")
    run = functools.partial(
        pl.kernel,
        mesh=mesh,
        compiler_params=pltpu.CompilerParams(needs_layout_passes=False),
        out_type=jax.ShapeDtypeStruct((NJT, NIT * 1024), jnp.float32),
        scratch_types=[
            pltpu.VMEM_SHARED((NTAB,), jnp.float32),
            pltpu.VMEM((2, 8, IT_PER_W * 256), jnp.float32),
            pltpu.VMEM((2, ROWS, 128), jnp.int32),
            pltpu.VMEM((2, CHUNK), jnp.float32),
            pltpu.VMEM((WINDOW,), jnp.float32),
            pltpu.SMEM((4,), jnp.int32),
            pltpu.SMEM((2,), jnp.int32),
            pltpu.SMEM((2,), jnp.int32),
            pltpu.SemaphoreType.DMA,
            pltpu.SemaphoreType.DMA,
            pltpu.SemaphoreType.DMA,
        ],
    )(_sc_kernel)
    inp_phys = (
        inp.reshape(NIT, 128, L, 2).transpose(2, 0, 3, 1).reshape(L, NIT * 256)
    )
    out_phys = run(inp_phys, data.reshape(-1))
    return (
        out_phys.reshape(NJT, NIT, 8, 128).transpose(1, 3, 0, 2).reshape(B, L)
    )


# speedup vs baseline: 254.4701x; 1.4209x over previous
import functools

import jax
import jax.numpy as jnp
import numpy as np
from jax import lax
from jax.experimental import pallas as pl
from jax.experimental.pallas import tpu as pltpu
from jax.experimental.pallas import tpu_sc as plsc

B = 16384
L = 200
TABLE = 1024
N = B * L

NUM_CORES = 2
NUM_SUBCORES = 16
NW = NUM_CORES * NUM_SUBCORES

NJT = L // 8
NIT = B // 128
IT_PER_W = NIT // NW
CHUNK = IT_PER_W * 1024
ROWS = CHUNK // 128
GROUPS = CHUNK // 16

NTAB = TABLE * TABLE
WINDOW = 32768
I32MIN = np.int32(-2147483648)
I32MAX = np.int32(2147483647)

B0 = np.float32(1.2)
B1 = np.float32(0.07)
M0 = np.float32(1023.9999 / (0.6 - (-1.2)))
M1 = np.float32(1023.9999 / (2 * 0.07))


def _sc_kernel(
    inp_hbm, tab_hbm, out_hbm, tab_sh, inp_v, idx_v, val_v, cache_v,
    mflag, wstate, used_dma, isem, gsem, osem
):
    cid = lax.axis_index("c")
    sid = lax.axis_index("s")
    wid = sid * NUM_CORES + cid
    it0 = wid * IT_PER_W

    SHARD = NTAB // NUM_SUBCORES
    pltpu.sync_copy(
        tab_hbm.at[pl.ds(sid * SHARD, SHARD)],
        tab_sh.at[pl.ds(sid * SHARD, SHARD)],
    )
    wstate[0] = 0
    wstate[1] = 0
    plsc.subcore_barrier()

    def inp_src(c):
        return inp_hbm.at[pl.ds(c * 8, 8), wid]

    def out_dst(c):
        return out_hbm.at[pl.ds(c * 8, 8), pl.ds(it0 * 128, IT_PER_W * 128)]

    def start_inp(c):
        pltpu.async_copy(inp_src(c), inp_v.at[c & 1], isem)

    def wait_inp(c):
        pltpu.make_async_copy(inp_src(c), inp_v.at[c & 1], isem).wait()

    def compute(c):
        p = c & 1

        def grp(g, carry2):
            mn, mx = carry2
            js = g // 32
            itl = (g // 8) % 4
            col = (g % 8) * 16
            x = inp_v[p, js, 2 * itl, pl.ds(col, 16)]
            y = inp_v[p, js, 2 * itl + 1, pl.ds(col, 16)]
            xi = ((x + B0) * M0).astype(jnp.int32)
            yi = ((y + B1) * M1).astype(jnp.int32)
            iv = xi * TABLE + yi
            idx_v[p, g // 8, pl.ds(col, 16)] = iv
            return jnp.minimum(mn, iv), jnp.maximum(mx, iv)

        mn, mx = lax.fori_loop(
            0,
            GROUPS,
            grp,
            (jnp.full((16,), I32MAX, jnp.int32),
             jnp.full((16,), I32MIN, jnp.int32)),
            unroll=4,
        )
        mflag[2 * p] = -lax.reduce_max(-mn, (0,))
        mflag[2 * p + 1] = lax.reduce_max(mx, (0,))

    def gather_chunk(c):
        p = c & 1
        mn = mflag[2 * p]
        mx = mflag[2 * p + 1]
        base_cur = wstate[0]
        fits_cur = jnp.logical_and(
            wstate[1] == 1,
            jnp.logical_and(mn >= base_cur, mx < base_cur + WINDOW),
        )
        new_base = pl.multiple_of(
            jnp.clip((mn >> 3) << 3, 0, NTAB - WINDOW), 8
        )
        fits_new = jnp.logical_and(mn >= new_base, mx < new_base + WINDOW)

        def restage():
            pltpu.sync_copy(tab_sh.at[pl.ds(new_base, WINDOW)], cache_v)
            wstate[0] = new_base
            wstate[1] = 1

        pl.when(jnp.logical_and(jnp.logical_not(fits_cur), fits_new))(restage)

        def local_gather():
            b = wstate[0]

            def lg(g, carry2):
                col = (g % 8) * 16
                iv = idx_v[p, g // 8, pl.ds(col, 16)] - b
                val_v[p, g // 32, pl.ds((g % 32) * 16, 16)] = (
                    plsc.load_gather(cache_v, [iv])
                )
                return carry2

            lax.fori_loop(0, GROUPS, lg, 0, unroll=4)
            used_dma[p] = 0

        def dma_gather():
            for j in range(ROWS):
                pltpu.async_copy(
                    tab_sh.at[idx_v.at[p, j]],
                    val_v.at[p, j // 4, pl.ds((j % 4) * 128, 128)],
                    gsem,
                )
            used_dma[p] = 1

        use_local = jnp.logical_or(fits_cur, fits_new)
        pl.when(use_local)(local_gather)
        pl.when(jnp.logical_not(use_local))(dma_gather)

    def wait_gathers(c):
        pl.when(used_dma[c & 1] == 1)(
            lambda: pltpu.make_async_copy(
                out_dst(c), val_v.at[c & 1], gsem
            ).wait()
        )

    def start_out(c):
        pltpu.async_copy(val_v.at[c & 1], out_dst(c), osem)

    def wait_out(c):
        pltpu.make_async_copy(val_v.at[c & 1], out_dst(c), osem).wait()

    start_inp(0)

    def chunk_body(c, carry):
        pl.when(c < NJT)(lambda: wait_inp(c))
        pl.when(c + 1 < NJT)(lambda: start_inp(c + 1))
        pl.when(c < NJT)(lambda: compute(c))
        pl.when((c >= 1) & (c <= NJT))(lambda: wait_gathers(c - 1))
        pl.when((c >= 1) & (c <= NJT))(lambda: start_out(c - 1))
        pl.when((c >= 2) & (c <= NJT + 1))(lambda: wait_out(c - 2))
        pl.when(c < NJT)(lambda: gather_chunk(c))
        return carry

    lax.fori_loop(0, NJT + 2, chunk_body, 0)


@jax.jit
def kernel(inp, data):
    mesh = plsc.VectorSubcoreMesh(core_axis_name="c", subcore_axis_name="s")
    run = functools.partial(
        pl.kernel,
        mesh=mesh,
        compiler_params=pltpu.CompilerParams(needs_layout_passes=False),
        out_type=jax.ShapeDtypeStruct((L, B), jnp.float32),
        scratch_types=[
            pltpu.VMEM_SHARED((NTAB,), jnp.float32),
            pltpu.VMEM((2, 8, 8, 128), jnp.float32),
            pltpu.VMEM((2, ROWS, 128), jnp.int32),
            pltpu.VMEM((2, 8, IT_PER_W * 128), jnp.float32),
            pltpu.VMEM((WINDOW,), jnp.float32),
            pltpu.SMEM((4,), jnp.int32),
            pltpu.SMEM((2,), jnp.int32),
            pltpu.SMEM((2,), jnp.int32),
            pltpu.SemaphoreType.DMA,
            pltpu.SemaphoreType.DMA,
            pltpu.SemaphoreType.DMA,
        ],
    )(_sc_kernel)
    inp_phys = (
        inp.reshape(NIT, 128, L, 2)
        .transpose(2, 0, 3, 1)
        .reshape(L, NIT // 4, 8, 128)
    )
    return run(inp_phys, data.reshape(-1)).T


# speedup vs baseline: 285.1274x; 1.1205x over previous
import functools

import jax
import jax.numpy as jnp
import numpy as np
from jax import lax
from jax.experimental import pallas as pl
from jax.experimental.pallas import tpu as pltpu
from jax.experimental.pallas import tpu_sc as plsc

B = 16384
L = 200
TABLE = 1024
N = B * L

NUM_CORES = 2
NUM_SUBCORES = 16
NW = NUM_CORES * NUM_SUBCORES

NJT = L // 8
NIT = B // 128
IT_PER_W = NIT // NW
CHUNK = IT_PER_W * 1024
ROWS = CHUNK // 128
GROUPS = CHUNK // 16

NTAB = TABLE * TABLE
WINDOW = 32768
I32MIN = np.int32(-2147483648)
I32MAX = np.int32(2147483647)

B0 = np.float32(1.2)
B1 = np.float32(0.07)
M0 = np.float32(1023.9999 / (0.6 - (-1.2)))
M1 = np.float32(1023.9999 / (2 * 0.07))


def _sc_kernel(
    inp_hbm, tab_hbm, out_hbm, tab_sh, inp_v, idx_v, val_v, cache_v,
    wstate, used_dma, tsem, isem, gsem, osem
):
    cid = lax.axis_index("c")
    sid = lax.axis_index("s")
    wid = sid * NUM_CORES + cid
    it0 = wid * IT_PER_W

    SHARD = NTAB // NUM_SUBCORES
    pltpu.async_copy(
        tab_hbm.at[pl.ds(sid * SHARD, SHARD)],
        tab_sh.at[pl.ds(sid * SHARD, SHARD)],
        tsem,
    )
    wstate[0] = 0
    wstate[1] = 0

    def wait_table():
        pltpu.make_async_copy(
            tab_hbm.at[pl.ds(sid * SHARD, SHARD)],
            tab_sh.at[pl.ds(sid * SHARD, SHARD)],
            tsem,
        ).wait()
        plsc.subcore_barrier()

    def inp_src(c):
        return inp_hbm.at[pl.ds(c * 8, 8), wid]

    def out_dst(c):
        return out_hbm.at[pl.ds(c * 8, 8), pl.ds(it0 * 128, IT_PER_W * 128)]

    def start_inp(c):
        pltpu.async_copy(inp_src(c), inp_v.at[c & 1], isem)

    def wait_inp(c):
        pltpu.make_async_copy(inp_src(c), inp_v.at[c & 1], isem).wait()

    def xy(p, g):
        js = g // 32
        itl = (g // 8) % 4
        col = (g % 8) * 16
        x = inp_v[p, js, 2 * itl, pl.ds(col, 16)]
        y = inp_v[p, js, 2 * itl + 1, pl.ds(col, 16)]
        xi = ((x + B0) * M0).astype(jnp.int32)
        yi = ((y + B1) * M1).astype(jnp.int32)
        return xi * TABLE + yi

    def fused(c):
        p = c & 1
        b = wstate[0]

        def grp(g, carry2):
            mn, mx = carry2
            ivb = xy(p, g) - b
            val_v[p, g // 32, pl.ds((g % 32) * 16, 16)] = plsc.load_gather(
                cache_v, [ivb & (WINDOW - 1)]
            )
            return jnp.minimum(mn, ivb), jnp.maximum(mx, ivb)

        mn, mx = lax.fori_loop(
            0,
            GROUPS,
            grp,
            (jnp.full((16,), I32MAX, jnp.int32),
             jnp.full((16,), I32MIN, jnp.int32)),
            unroll=4,
        )
        mn = -lax.reduce_max(-mn, (0,))
        mx = lax.reduce_max(mx, (0,))
        used_dma[p] = 0
        ok = jnp.logical_and(
            wstate[1] == 1,
            jnp.logical_and(mn >= 0, mx < WINDOW),
        )

        def fallback():
            tmn = mn + b
            tmx = mx + b
            new_base = pl.multiple_of(
                jnp.clip((tmn >> 3) << 3, 0, NTAB - WINDOW), 8
            )
            fits = jnp.logical_and(
                tmn >= new_base, tmx < new_base + WINDOW
            )

            def local_redo():
                pltpu.sync_copy(tab_sh.at[pl.ds(new_base, WINDOW)], cache_v)
                wstate[0] = new_base
                wstate[1] = 1

                def lg(g, carry):
                    ivb = xy(p, g) - new_base
                    val_v[p, g // 32, pl.ds((g % 32) * 16, 16)] = (
                        plsc.load_gather(cache_v, [ivb])
                    )
                    return carry

                lax.fori_loop(0, GROUPS, lg, 0, unroll=4)

            def dma_redo():
                def cg(g, carry):
                    col = (g % 8) * 16
                    idx_v[g // 8, pl.ds(col, 16)] = xy(p, g)
                    return carry

                lax.fori_loop(0, GROUPS, cg, 0, unroll=4)
                for j in range(ROWS):
                    pltpu.async_copy(
                        tab_sh.at[idx_v.at[j]],
                        val_v.at[p, j // 4, pl.ds((j % 4) * 128, 128)],
                        gsem,
                    )
                used_dma[p] = 1

            pl.when(fits)(local_redo)
            pl.when(jnp.logical_not(fits))(dma_redo)

        pl.when(jnp.logical_not(ok))(fallback)

    def wait_gathers(c):
        pl.when(used_dma[c & 1] == 1)(
            lambda: pltpu.make_async_copy(
                out_dst(c), val_v.at[c & 1], gsem
            ).wait()
        )

    def start_out(c):
        pltpu.async_copy(val_v.at[c & 1], out_dst(c), osem)

    def wait_out(c):
        pltpu.make_async_copy(val_v.at[c & 1], out_dst(c), osem).wait()

    start_inp(0)

    def chunk_body(c, carry):
        pl.when(c < NJT)(lambda: wait_inp(c))
        pl.when(c + 1 < NJT)(lambda: start_inp(c + 1))
        pl.when((c >= 1) & (c <= NJT))(lambda: wait_gathers(c - 1))
        pl.when((c >= 1) & (c <= NJT))(lambda: start_out(c - 1))
        pl.when((c >= 2) & (c <= NJT + 1))(lambda: wait_out(c - 2))
        pl.when(c == 0)(wait_table)
        pl.when(c < NJT)(lambda: fused(c))
        return carry

    lax.fori_loop(0, NJT + 2, chunk_body, 0)


@jax.jit
def kernel(inp, data):
    mesh = plsc.VectorSubcoreMesh(core_axis_name="c", subcore_axis_name="s")
    run = functools.partial(
        pl.kernel,
        mesh=mesh,
        compiler_params=pltpu.CompilerParams(needs_layout_passes=False),
        out_type=jax.ShapeDtypeStruct((L, B), jnp.float32),
        scratch_types=[
            pltpu.VMEM_SHARED((NTAB,), jnp.float32),
            pltpu.VMEM((2, 8, 8, 128), jnp.float32),
            pltpu.VMEM((ROWS, 128), jnp.int32),
            pltpu.VMEM((2, 8, IT_PER_W * 128), jnp.float32),
            pltpu.VMEM((WINDOW,), jnp.float32),
            pltpu.SMEM((2,), jnp.int32),
            pltpu.SMEM((2,), jnp.int32),
            pltpu.SemaphoreType.DMA,
            pltpu.SemaphoreType.DMA,
            pltpu.SemaphoreType.DMA,
            pltpu.SemaphoreType.DMA,
        ],
    )(_sc_kernel)
    inp_phys = (
        inp.reshape(NIT, 128, L, 2)
        .transpose(2, 0, 3, 1)
        .reshape(L, NIT // 4, 8, 128)
    )
    return run(inp_phys, data.reshape(-1)).T


# speedup vs baseline: 293.8618x; 1.0306x over previous
import functools

import jax
import jax.numpy as jnp
import numpy as np
from jax import lax
from jax.experimental import pallas as pl
from jax.experimental.pallas import tpu as pltpu
from jax.experimental.pallas import tpu_sc as plsc

B = 16384
L = 200
TABLE = 1024
N = B * L

NUM_CORES = 2
NUM_SUBCORES = 16
NW = NUM_CORES * NUM_SUBCORES

NJT = L // 8
NIT = B // 128
IT_PER_W = NIT // NW
CHUNK = IT_PER_W * 1024
ROWS = CHUNK // 128
GROUPS = CHUNK // 16

NTAB = TABLE * TABLE
WINDOW = 32768
I32MIN = np.int32(-2147483648)
I32MAX = np.int32(2147483647)

B0 = np.float32(1.2)
B1 = np.float32(0.07)
M0 = np.float32(1023.9999 / (0.6 - (-1.2)))
M1 = np.float32(1023.9999 / (2 * 0.07))


def _sc_kernel(
    inp_hbm, tab_hbm, out_hbm, tab_sh, inp_v, idx_v, val_v, cache_v,
    wstate, used_dma, tsem, isem, gsem, osem
):
    cid = lax.axis_index("c")
    sid = lax.axis_index("s")
    wid = sid * NUM_CORES + cid
    it0 = wid * IT_PER_W

    SHARD = NTAB // NUM_SUBCORES
    pltpu.async_copy(
        tab_hbm.at[pl.ds(sid * SHARD, SHARD)],
        tab_sh.at[pl.ds(sid * SHARD, SHARD)],
        tsem,
    )
    wstate[0] = 0
    wstate[1] = 0

    def wait_table():
        pltpu.make_async_copy(
            tab_hbm.at[pl.ds(sid * SHARD, SHARD)],
            tab_sh.at[pl.ds(sid * SHARD, SHARD)],
            tsem,
        ).wait()
        plsc.subcore_barrier()

    def inp_src(c):
        return inp_hbm.at[pl.ds(c * 8, 8), wid]

    def out_dst(c):
        return out_hbm.at[pl.ds(c * 8, 8), pl.ds(it0 * 128, IT_PER_W * 128)]

    def start_inp(c):
        pltpu.async_copy(inp_src(c), inp_v.at[c & 1], isem)

    def wait_inp(c):
        pltpu.make_async_copy(inp_src(c), inp_v.at[c & 1], isem).wait()

    def xy(p, g):
        js = g // 32
        itl = (g // 8) % 4
        col = (g % 8) * 16
        x = inp_v[p, js, 2 * itl, pl.ds(col, 16)]
        y = inp_v[p, js, 2 * itl + 1, pl.ds(col, 16)]
        xi = ((x + B0) * M0).astype(jnp.int32)
        yi = ((y + B1) * M1).astype(jnp.int32)
        return xi * TABLE + yi

    def fused(c):
        p = c & 1
        b = wstate[0]

        def grp(g, carry2):
            mn, mx = carry2
            ivb = xy(p, g) - b
            val_v[p, g // 32, pl.ds((g % 32) * 16, 16)] = plsc.load_gather(
                cache_v, [ivb & (WINDOW - 1)]
            )
            return jnp.minimum(mn, ivb), jnp.maximum(mx, ivb)

        mn, mx = lax.fori_loop(
            0,
            GROUPS,
            grp,
            (jnp.full((16,), I32MAX, jnp.int32),
             jnp.full((16,), I32MIN, jnp.int32)),
            unroll=8,
        )
        mn = -lax.reduce_max(-mn, (0,))
        mx = lax.reduce_max(mx, (0,))
        used_dma[p] = 0
        ok = jnp.logical_and(
            wstate[1] == 1,
            jnp.logical_and(mn >= 0, mx < WINDOW),
        )

        def fallback():
            tmn = mn + b
            tmx = mx + b
            new_base = pl.multiple_of(
                jnp.clip((tmn >> 3) << 3, 0, NTAB - WINDOW), 8
            )
            fits = jnp.logical_and(
                tmn >= new_base, tmx < new_base + WINDOW
            )

            def local_redo():
                pltpu.sync_copy(tab_sh.at[pl.ds(new_base, WINDOW)], cache_v)
                wstate[0] = new_base
                wstate[1] = 1

                def lg(g, carry):
                    ivb = xy(p, g) - new_base
                    val_v[p, g // 32, pl.ds((g % 32) * 16, 16)] = (
                        plsc.load_gather(cache_v, [ivb])
                    )
                    return carry

                lax.fori_loop(0, GROUPS, lg, 0, unroll=4)

            def dma_redo():
                def cg(g, carry):
                    col = (g % 8) * 16
                    idx_v[g // 8, pl.ds(col, 16)] = xy(p, g)
                    return carry

                lax.fori_loop(0, GROUPS, cg, 0, unroll=4)
                for j in range(ROWS):
                    pltpu.async_copy(
                        tab_sh.at[idx_v.at[j]],
                        val_v.at[p, j // 4, pl.ds((j % 4) * 128, 128)],
                        gsem,
                    )
                used_dma[p] = 1

            pl.when(fits)(local_redo)
            pl.when(jnp.logical_not(fits))(dma_redo)

        pl.when(jnp.logical_not(ok))(fallback)

    def wait_gathers(c):
        pl.when(used_dma[c & 1] == 1)(
            lambda: pltpu.make_async_copy(
                out_dst(c), val_v.at[c & 1], gsem
            ).wait()
        )

    def start_out(c):
        pltpu.async_copy(val_v.at[c & 1], out_dst(c), osem)

    def wait_out(c):
        pltpu.make_async_copy(val_v.at[c & 1], out_dst(c), osem).wait()

    start_inp(0)

    def chunk_body(c, carry):
        pl.when(c < NJT)(lambda: wait_inp(c))
        pl.when(c + 1 < NJT)(lambda: start_inp(c + 1))
        pl.when((c >= 1) & (c <= NJT))(lambda: wait_gathers(c - 1))
        pl.when((c >= 1) & (c <= NJT))(lambda: start_out(c - 1))
        pl.when((c >= 2) & (c <= NJT + 1))(lambda: wait_out(c - 2))
        pl.when(c == 0)(wait_table)
        pl.when(c < NJT)(lambda: fused(c))
        return carry

    lax.fori_loop(0, NJT + 2, chunk_body, 0)


@jax.jit
def kernel(inp, data):
    mesh = plsc.VectorSubcoreMesh(core_axis_name="c", subcore_axis_name="s")
    run = functools.partial(
        pl.kernel,
        mesh=mesh,
        compiler_params=pltpu.CompilerParams(needs_layout_passes=False),
        out_type=jax.ShapeDtypeStruct((L, B), jnp.float32),
        scratch_types=[
            pltpu.VMEM_SHARED((NTAB,), jnp.float32),
            pltpu.VMEM((2, 8, 8, 128), jnp.float32),
            pltpu.VMEM((ROWS, 128), jnp.int32),
            pltpu.VMEM((2, 8, IT_PER_W * 128), jnp.float32),
            pltpu.VMEM((WINDOW,), jnp.float32),
            pltpu.SMEM((2,), jnp.int32),
            pltpu.SMEM((2,), jnp.int32),
            pltpu.SemaphoreType.DMA,
            pltpu.SemaphoreType.DMA,
            pltpu.SemaphoreType.DMA,
            pltpu.SemaphoreType.DMA,
        ],
    )(_sc_kernel)
    inp_phys = (
        inp.reshape(NIT, 128, L, 2)
        .transpose(2, 0, 3, 1)
        .reshape(L, NIT // 4, 8, 128)
    )
    return run(inp_phys, data.reshape(-1)).T


# speedup vs baseline: 295.2827x; 1.0048x over previous
import functools

import jax
import jax.numpy as jnp
import numpy as np
from jax import lax
from jax.experimental import pallas as pl
from jax.experimental.pallas import tpu as pltpu
from jax.experimental.pallas import tpu_sc as plsc

B = 16384
L = 200
TABLE = 1024
N = B * L

NUM_CORES = 2
NUM_SUBCORES = 16
NW = NUM_CORES * NUM_SUBCORES

NJT = L // 8
NIT = B // 128
IT_PER_W = NIT // NW
CHUNK = IT_PER_W * 1024
ROWS = CHUNK // 128
GROUPS = CHUNK // 16

NTAB = TABLE * TABLE
WINDOW = 32768
I32MIN = np.int32(-2147483648)
I32MAX = np.int32(2147483647)

B0 = np.float32(1.2)
B1 = np.float32(0.07)
M0 = np.float32(1023.9999 / (0.6 - (-1.2)))
M1 = np.float32(1023.9999 / (2 * 0.07))


def _sc_kernel(
    inp_hbm, tab_hbm, out_hbm, tab_sh, inp_v, idx_v, val_v, cache_v,
    wstate, used_dma, tsem, isem, gsem, osem
):
    cid = lax.axis_index("c")
    sid = lax.axis_index("s")
    wid = sid * NUM_CORES + cid
    it0 = wid * IT_PER_W

    SHARD = NTAB // NUM_SUBCORES
    pltpu.async_copy(
        tab_hbm.at[pl.ds(sid * SHARD, SHARD)],
        tab_sh.at[pl.ds(sid * SHARD, SHARD)],
        tsem,
    )
    wstate[0] = 0
    wstate[1] = 0

    def wait_table():
        pltpu.make_async_copy(
            tab_hbm.at[pl.ds(sid * SHARD, SHARD)],
            tab_sh.at[pl.ds(sid * SHARD, SHARD)],
            tsem,
        ).wait()
        plsc.subcore_barrier()

    def inp_src(c):
        return inp_hbm.at[pl.ds(c * 8, 8), wid]

    def out_dst(c):
        return out_hbm.at[pl.ds(c * 8, 8), pl.ds(it0 * 128, IT_PER_W * 128)]

    def start_inp(c):
        pltpu.async_copy(inp_src(c), inp_v.at[c & 1], isem)

    def wait_inp(c):
        pltpu.make_async_copy(inp_src(c), inp_v.at[c & 1], isem).wait()

    def xy(p, g):
        js = g // 32
        itl = (g // 8) % 4
        col = (g % 8) * 16
        x = inp_v[p, js, 2 * itl, pl.ds(col, 16)]
        y = inp_v[p, js, 2 * itl + 1, pl.ds(col, 16)]
        xi = ((x + B0) * M0).astype(jnp.int32)
        yi = ((y + B1) * M1).astype(jnp.int32)
        return xi * TABLE + yi

    def fused(c):
        p = c & 1
        b = wstate[0]

        def grp(g, carry2):
            mn, mx = carry2
            ivb = xy(p, g) - b
            val_v[p, g // 32, pl.ds((g % 32) * 16, 16)] = plsc.load_gather(
                cache_v, [ivb & (WINDOW - 1)]
            )
            return jnp.minimum(mn, ivb), jnp.maximum(mx, ivb)

        mn, mx = lax.fori_loop(
            0,
            GROUPS,
            grp,
            (jnp.full((16,), I32MAX, jnp.int32),
             jnp.full((16,), I32MIN, jnp.int32)),
            unroll=16,
        )
        mn = -lax.reduce_max(-mn, (0,))
        mx = lax.reduce_max(mx, (0,))
        used_dma[p] = 0
        ok = jnp.logical_and(
            wstate[1] == 1,
            jnp.logical_and(mn >= 0, mx < WINDOW),
        )

        def fallback():
            tmn = mn + b
            tmx = mx + b
            new_base = pl.multiple_of(
                jnp.clip((tmn >> 3) << 3, 0, NTAB - WINDOW), 8
            )
            fits = jnp.logical_and(
                tmn >= new_base, tmx < new_base + WINDOW
            )

            def local_redo():
                pltpu.sync_copy(tab_sh.at[pl.ds(new_base, WINDOW)], cache_v)
                wstate[0] = new_base
                wstate[1] = 1

                def lg(g, carry):
                    ivb = xy(p, g) - new_base
                    val_v[p, g // 32, pl.ds((g % 32) * 16, 16)] = (
                        plsc.load_gather(cache_v, [ivb])
                    )
                    return carry

                lax.fori_loop(0, GROUPS, lg, 0, unroll=4)

            def dma_redo():
                def cg(g, carry):
                    col = (g % 8) * 16
                    idx_v[g // 8, pl.ds(col, 16)] = xy(p, g)
                    return carry

                lax.fori_loop(0, GROUPS, cg, 0, unroll=4)
                for j in range(ROWS):
                    pltpu.async_copy(
                        tab_sh.at[idx_v.at[j]],
                        val_v.at[p, j // 4, pl.ds((j % 4) * 128, 128)],
                        gsem,
                    )
                used_dma[p] = 1

            pl.when(fits)(local_redo)
            pl.when(jnp.logical_not(fits))(dma_redo)

        pl.when(jnp.logical_not(ok))(fallback)

    def wait_gathers(c):
        pl.when(used_dma[c & 1] == 1)(
            lambda: pltpu.make_async_copy(
                out_dst(c), val_v.at[c & 1], gsem
            ).wait()
        )

    def start_out(c):
        pltpu.async_copy(val_v.at[c & 1], out_dst(c), osem)

    def wait_out(c):
        pltpu.make_async_copy(val_v.at[c & 1], out_dst(c), osem).wait()

    start_inp(0)

    def chunk_body(c, carry):
        pl.when(c < NJT)(lambda: wait_inp(c))
        pl.when(c + 1 < NJT)(lambda: start_inp(c + 1))
        pl.when((c >= 1) & (c <= NJT))(lambda: wait_gathers(c - 1))
        pl.when((c >= 1) & (c <= NJT))(lambda: start_out(c - 1))
        pl.when((c >= 2) & (c <= NJT + 1))(lambda: wait_out(c - 2))
        pl.when(c == 0)(wait_table)
        pl.when(c < NJT)(lambda: fused(c))
        return carry

    lax.fori_loop(0, NJT + 2, chunk_body, 0)


@jax.jit
def kernel(inp, data):
    mesh = plsc.VectorSubcoreMesh(core_axis_name="c", subcore_axis_name="s")
    run = functools.partial(
        pl.kernel,
        mesh=mesh,
        compiler_params=pltpu.CompilerParams(needs_layout_passes=False),
        out_type=jax.ShapeDtypeStruct((L, B), jnp.float32),
        scratch_types=[
            pltpu.VMEM_SHARED((NTAB,), jnp.float32),
            pltpu.VMEM((2, 8, 8, 128), jnp.float32),
            pltpu.VMEM((ROWS, 128), jnp.int32),
            pltpu.VMEM((2, 8, IT_PER_W * 128), jnp.float32),
            pltpu.VMEM((WINDOW,), jnp.float32),
            pltpu.SMEM((2,), jnp.int32),
            pltpu.SMEM((2,), jnp.int32),
            pltpu.SemaphoreType.DMA,
            pltpu.SemaphoreType.DMA,
            pltpu.SemaphoreType.DMA,
            pltpu.SemaphoreType.DMA,
        ],
    )(_sc_kernel)
    inp_phys = (
        inp.reshape(NIT, 128, L, 2)
        .transpose(2, 0, 3, 1)
        .reshape(L, NIT // 4, 8, 128)
    )
    return run(inp_phys, data.reshape(-1)).T


# speedup vs baseline: 711.7042x; 2.4102x over previous
import functools

import jax
import jax.numpy as jnp
import numpy as np
from jax import lax
from jax.experimental import pallas as pl
from jax.experimental.pallas import tpu as pltpu
from jax.experimental.pallas import tpu_sc as plsc

B = 16384
L = 200
TABLE = 1024
N = B * L

NUM_CORES = 2
NUM_SUBCORES = 16
NW = NUM_CORES * NUM_SUBCORES

NJT = L // 8
NIT = B // 128
IT_PER_W = NIT // NW
CHUNK = IT_PER_W * 1024
ROWS = CHUNK // 128
GROUPS = CHUNK // 16

NTAB = TABLE * TABLE
WINDOW = 32768
I32MIN = np.int32(-2147483648)
I32MAX = np.int32(2147483647)

B0 = np.float32(1.2)
B1 = np.float32(0.07)
M0 = np.float32(1023.9999 / (0.6 - (-1.2)))
M1 = np.float32(1023.9999 / (2 * 0.07))


def _sc_kernel(
    inp_hbm, tab_hbm, out_hbm, tab_sh, inp_v, idx_v, val_v, cache_v, fill1,
    wstate, fillstate, used_dma, tsem, isem, gsem, osem
):
    cid = lax.axis_index("c")
    sid = lax.axis_index("s")
    wid = sid * NUM_CORES + cid
    it0 = wid * IT_PER_W

    SHARD = NTAB // NUM_SUBCORES
    pltpu.async_copy(
        tab_hbm.at[pl.ds(sid * SHARD, SHARD)],
        tab_sh.at[pl.ds(sid * SHARD, SHARD)],
        tsem,
    )
    wstate[0] = 0
    wstate[1] = 0
    fillstate[0] = -1
    fillstate[1] = -1

    def wait_table():
        pltpu.make_async_copy(
            tab_hbm.at[pl.ds(sid * SHARD, SHARD)],
            tab_sh.at[pl.ds(sid * SHARD, SHARD)],
            tsem,
        ).wait()
        plsc.subcore_barrier()

    def inp_src(c):
        return inp_hbm.at[pl.ds(c * 8, 8), wid]

    def out_dst(c):
        return out_hbm.at[pl.ds(c * 8, 8), pl.ds(it0 * 128, IT_PER_W * 128)]

    def start_inp(c):
        pltpu.async_copy(inp_src(c), inp_v.at[c & 1], isem)

    def wait_inp(c):
        pltpu.make_async_copy(inp_src(c), inp_v.at[c & 1], isem).wait()

    def xy(p, g):
        js = g // 32
        itl = (g // 8) % 4
        col = (g % 8) * 16
        x = inp_v[p, js, 2 * itl, pl.ds(col, 16)]
        y = inp_v[p, js, 2 * itl + 1, pl.ds(col, 16)]
        xi = ((x + B0) * M0).astype(jnp.int32)
        yi = ((y + B1) * M1).astype(jnp.int32)
        return xi * TABLE + yi

    def fused(c):
        p = c & 1

        def chk(g, carry4):
            xmn, xmx, ymn, ymx = carry4
            js = g // 32
            itl = (g // 8) % 4
            col = (g % 8) * 16
            x = inp_v[p, js, 2 * itl, pl.ds(col, 16)]
            y = inp_v[p, js, 2 * itl + 1, pl.ds(col, 16)]
            return (
                jnp.minimum(xmn, x),
                jnp.maximum(xmx, x),
                jnp.minimum(ymn, y),
                jnp.maximum(ymx, y),
            )

        big = jnp.full((16,), np.float32(3.0e38), jnp.float32)
        xmn, xmx, ymn, ymx = lax.fori_loop(
            0, GROUPS, chk, (big, -big, big, -big), unroll=16
        )
        xh = lax.reduce_max(xmx, (0,))
        xl = -lax.reduce_max(-xmn, (0,))
        yh = lax.reduce_max(ymx, (0,))
        yl = -lax.reduce_max(-ymn, (0,))
        uniform = jnp.logical_and(xl == xh, yl == yh)

        def unipath():
            xi = ((xh + B0) * M0).astype(jnp.int32)
            yi = ((yh + B1) * M1).astype(jnp.int32)
            iv = xi * TABLE + yi
            used_dma[p] = 0

            def refill():
                base8 = pl.multiple_of((iv >> 3) << 3, 8)
                pltpu.sync_copy(tab_sh.at[pl.ds(base8, 8)], fill1)
                v = fill1[iv & 7]

                def fg(g, carry):
                    val_v[p, g // 32, pl.ds((g % 32) * 16, 16)] = jnp.full(
                        (16,), v, jnp.float32
                    )
                    return carry

                lax.fori_loop(0, GROUPS, fg, 0, unroll=16)
                fillstate[p] = iv

            pl.when(fillstate[p] != iv)(refill)

        pl.when(uniform)(unipath)
        pl.when(jnp.logical_not(uniform))(lambda: general(c))

    def general(c):
        p = c & 1
        b = wstate[0]
        fillstate[p] = -1

        def grp(g, carry2):
            mn, mx = carry2
            ivb = xy(p, g) - b
            val_v[p, g // 32, pl.ds((g % 32) * 16, 16)] = plsc.load_gather(
                cache_v, [ivb & (WINDOW - 1)]
            )
            return jnp.minimum(mn, ivb), jnp.maximum(mx, ivb)

        mn, mx = lax.fori_loop(
            0,
            GROUPS,
            grp,
            (jnp.full((16,), I32MAX, jnp.int32),
             jnp.full((16,), I32MIN, jnp.int32)),
            unroll=16,
        )
        mn = -lax.reduce_max(-mn, (0,))
        mx = lax.reduce_max(mx, (0,))
        used_dma[p] = 0
        ok = jnp.logical_and(
            wstate[1] == 1,
            jnp.logical_and(mn >= 0, mx < WINDOW),
        )

        def fallback():
            tmn = mn + b
            tmx = mx + b
            new_base = pl.multiple_of(
                jnp.clip((tmn >> 3) << 3, 0, NTAB - WINDOW), 8
            )
            fits = jnp.logical_and(
                tmn >= new_base, tmx < new_base + WINDOW
            )

            def local_redo():
                pltpu.sync_copy(tab_sh.at[pl.ds(new_base, WINDOW)], cache_v)
                wstate[0] = new_base
                wstate[1] = 1

                def lg(g, carry):
                    ivb = xy(p, g) - new_base
                    val_v[p, g // 32, pl.ds((g % 32) * 16, 16)] = (
                        plsc.load_gather(cache_v, [ivb])
                    )
                    return carry

                lax.fori_loop(0, GROUPS, lg, 0, unroll=4)

            def dma_redo():
                def cg(g, carry):
                    col = (g % 8) * 16
                    idx_v[g // 8, pl.ds(col, 16)] = xy(p, g)
                    return carry

                lax.fori_loop(0, GROUPS, cg, 0, unroll=4)
                for j in range(ROWS):
                    pltpu.async_copy(
                        tab_sh.at[idx_v.at[j]],
                        val_v.at[p, j // 4, pl.ds((j % 4) * 128, 128)],
                        gsem,
                    )
                used_dma[p] = 1

            pl.when(fits)(local_redo)
            pl.when(jnp.logical_not(fits))(dma_redo)

        pl.when(jnp.logical_not(ok))(fallback)

    def wait_gathers(c):
        pl.when(used_dma[c & 1] == 1)(
            lambda: pltpu.make_async_copy(
                out_dst(c), val_v.at[c & 1], gsem
            ).wait()
        )

    def start_out(c):
        pltpu.async_copy(val_v.at[c & 1], out_dst(c), osem)

    def wait_out(c):
        pltpu.make_async_copy(val_v.at[c & 1], out_dst(c), osem).wait()

    start_inp(0)

    def chunk_body(c, carry):
        pl.when(c < NJT)(lambda: wait_inp(c))
        pl.when(c + 1 < NJT)(lambda: start_inp(c + 1))
        pl.when((c >= 1) & (c <= NJT))(lambda: wait_gathers(c - 1))
        pl.when((c >= 1) & (c <= NJT))(lambda: start_out(c - 1))
        pl.when((c >= 2) & (c <= NJT + 1))(lambda: wait_out(c - 2))
        pl.when(c == 0)(wait_table)
        pl.when(c < NJT)(lambda: fused(c))
        return carry

    lax.fori_loop(0, NJT + 2, chunk_body, 0)


@jax.jit
def kernel(inp, data):
    mesh = plsc.VectorSubcoreMesh(core_axis_name="c", subcore_axis_name="s")
    run = functools.partial(
        pl.kernel,
        mesh=mesh,
        compiler_params=pltpu.CompilerParams(needs_layout_passes=False),
        out_type=jax.ShapeDtypeStruct((L, B), jnp.float32),
        scratch_types=[
            pltpu.VMEM_SHARED((NTAB,), jnp.float32),
            pltpu.VMEM((2, 8, 8, 128), jnp.float32),
            pltpu.VMEM((ROWS, 128), jnp.int32),
            pltpu.VMEM((2, 8, IT_PER_W * 128), jnp.float32),
            pltpu.VMEM((WINDOW,), jnp.float32),
            pltpu.SMEM((8,), jnp.float32),
            pltpu.SMEM((2,), jnp.int32),
            pltpu.SMEM((2,), jnp.int32),
            pltpu.SMEM((2,), jnp.int32),
            pltpu.SemaphoreType.DMA,
            pltpu.SemaphoreType.DMA,
            pltpu.SemaphoreType.DMA,
            pltpu.SemaphoreType.DMA,
        ],
    )(_sc_kernel)
    inp_phys = (
        inp.reshape(NIT, 128, L, 2)
        .transpose(2, 0, 3, 1)
        .reshape(L, NIT // 4, 8, 128)
    )
    return run(inp_phys, data.reshape(-1)).T
